# baseline (speedup 1.0000x reference)
import functools

import jax
import jax.numpy as jnp
from jax import lax
from jax.experimental import pallas as pl
from jax.experimental.pallas import tpu as pltpu
from jax.experimental.pallas import tpu_sc as plsc

NUM_FRAMES = 32
T = 300
IDX = [(i * (T - 1)) // (NUM_FRAMES - 1) for i in range(NUM_FRAMES)]

SC_FRAMES = 8
TC_FRAMES = NUM_FRAMES - SC_FRAMES

CROWS = 56
CPP = 224 // CROWS
CPF = 3 * CPP

_info = plsc.get_sparse_core_info()
_NC, _NS = _info.num_cores, _info.num_subcores
NW = _NC * _NS
CPT = SC_FRAMES * CPF // NW
NBUF = min(CPT, 8)


def _sc_gather_kernel(frames_hbm, out_hbm, *scratch):
    bufs = scratch[:NBUF]
    sins = scratch[NBUF:2 * NBUF]
    souts = scratch[2 * NBUF:]

    wid = lax.axis_index("s") * _NC + lax.axis_index("c")

    def coords(c):
        g = wid * CPT + c
        f = g // CPF
        ch, r = (g % CPF) // CPP, ((g % CPF) % CPP) * CROWS
        src = (f + TC_FRAMES) * (T - 1) // (NUM_FRAMES - 1)
        return f, src, ch, r

    def in_copy(c):
        f, src, ch, r = coords(c)
        return pltpu.make_async_copy(
            frames_hbm.at[src, ch, pl.ds(r, CROWS)], bufs[c % NBUF], sins[c % NBUF]
        )

    def out_copy(c):
        f, src, ch, r = coords(c)
        return pltpu.make_async_copy(
            bufs[c % NBUF], out_hbm.at[f, ch, pl.ds(r, CROWS)], souts[c % NBUF]
        )

    for c in range(min(NBUF, CPT)):
        in_copy(c).start()
    for c in range(CPT):
        in_copy(c).wait()
        out_copy(c).start()
        if c + NBUF < CPT:
            out_copy(c).wait()
            in_copy(c + NBUF).start()
    for c in range(max(0, CPT - NBUF), CPT):
        out_copy(c).wait()


def _tc_copy_kernel(frames_ref, out_ref):
    out_ref[...] = frames_ref[...]


@jax.jit
def kernel(frames):
    mesh = plsc.VectorSubcoreMesh(core_axis_name="c", subcore_axis_name="s")
    sc_out = pl.kernel(
        _sc_gather_kernel,
        out_type=jax.ShapeDtypeStruct((SC_FRAMES, 3, 224, 224), jnp.float32),
        mesh=mesh,
        scratch_types=(
            [pltpu.VMEM((CROWS, 224), jnp.float32)] * NBUF
            + [pltpu.SemaphoreType.DMA] * (2 * NBUF)
        ),
    )(frames)

    tc_out = pl.pallas_call(
        _tc_copy_kernel,
        grid=(TC_FRAMES,),
        in_specs=[
            pl.BlockSpec(
                (1, 3, 224, 224),
                lambda i: ((i * (T - 1)) // (NUM_FRAMES - 1), 0, 0, 0),
            )
        ],
        out_specs=pl.BlockSpec((1, 3, 224, 224), lambda i: (i, 0, 0, 0)),
        out_shape=jax.ShapeDtypeStruct((NUM_FRAMES, 3, 224, 224), jnp.float32),
        compiler_params=pltpu.CompilerParams(
            dimension_semantics=("arbitrary",),
        ),
    )(frames)

    return lax.dynamic_update_slice(tc_out, sc_out, (TC_FRAMES, 0, 0, 0))
